# baseline (speedup 1.0000x reference)
import functools

import jax
import jax.numpy as jnp
from jax import lax
from jax.experimental import pallas as pl
from jax.experimental.pallas import tpu as pltpu
from jax.experimental.pallas import tpu_sc as plsc

_B = 16384
_D = 64
_NC = 2
_NS = 16
_NW = _NC * _NS
_BPW = _B // _NW

_CHUNK = 128
_NCH = _BPW // _CHUNK

_BLK = 2048


def _gather_body(uidx_hbm, iidx_hbm, uemb_hbm, iemb_hbm, uout_hbm, iout_hbm,
                 uidx_v, iidx_v, urows_v, irows_v, sem_u, sem_i):
    wid = lax.axis_index("s") * _NC + lax.axis_index("c")
    base = wid * _BPW
    pltpu.sync_copy(uidx_hbm.at[pl.ds(wid * _NCH, _NCH)], uidx_v)
    pltpu.sync_copy(iidx_hbm.at[pl.ds(wid * _NCH, _NCH)], iidx_v)
    copies = []
    for k in range(_NCH):
        copies.append(pltpu.async_copy(
            uemb_hbm.at[uidx_v.at[k]],
            urows_v.at[pl.ds(k * _CHUNK, _CHUNK)], sem_u))
        copies.append(pltpu.async_copy(
            iemb_hbm.at[iidx_v.at[k]],
            irows_v.at[pl.ds(k * _CHUNK, _CHUNK)], sem_i))
    for c in copies:
        c.wait()
    pltpu.sync_copy(urows_v, uout_hbm.at[pl.ds(base, _BPW)])
    pltpu.sync_copy(irows_v, iout_hbm.at[pl.ds(base, _BPW)])


@functools.cache
def _gather():
    return pl.kernel(
        _gather_body,
        out_type=(jax.ShapeDtypeStruct((_B, _D), jnp.bfloat16),
                  jax.ShapeDtypeStruct((_B, _D), jnp.bfloat16)),
        mesh=plsc.VectorSubcoreMesh(core_axis_name="c", subcore_axis_name="s",
                                    num_cores=_NC, num_subcores=_NS),
        compiler_params=pltpu.CompilerParams(use_tc_tiling_on_sc=False),
        scratch_types=[
            pltpu.VMEM((_NCH, _CHUNK), jnp.int32),
            pltpu.VMEM((_NCH, _CHUNK), jnp.int32),
            pltpu.VMEM((_BPW, _D), jnp.bfloat16),
            pltpu.VMEM((_BPW, _D), jnp.bfloat16),
            pltpu.SemaphoreType.DMA,
            pltpu.SemaphoreType.DMA,
        ],
    )


def _mlp_body(u_ref, i_ref, w1u_ref, w1i_ref, b1_ref, w2_ref, b2_ref,
              w3t_ref, b3_ref, o_ref):
    u = u_ref[...]
    it = i_ref[...]
    b1 = b1_ref[...]
    outs = []
    for half in (slice(0, _D), slice(_D, 2 * _D)):
        h1 = jnp.dot(u[:, half], w1u_ref[...],
                     preferred_element_type=jnp.float32)
        h1 += jnp.dot(it[:, half], w1i_ref[...],
                      preferred_element_type=jnp.float32)
        h1 = jnp.maximum(h1 + b1, 0.0).astype(jnp.bfloat16)
        h2 = jnp.dot(h1, w2_ref[...], preferred_element_type=jnp.float32)
        h2 = jnp.maximum(h2 + b2_ref[...], 0.0)
        logit = jnp.sum(h2 * w3t_ref[...], axis=1, keepdims=True) + b3_ref[...]
        outs.append(1.0 / (1.0 + jnp.exp(-logit)))
    o_ref[...] = jnp.concatenate(outs, axis=1)


def _mlp(u2, i2, w1u, w1i, b1, w2, b2, w3t, b3):
    full = lambda s: pl.BlockSpec(s, lambda n: (0, 0))
    return pl.pallas_call(
        _mlp_body,
        grid=(_B // 2 // _BLK,),
        in_specs=[
            pl.BlockSpec((_BLK, 128), lambda n: (n, 0)),
            pl.BlockSpec((_BLK, 128), lambda n: (n, 0)),
            full((_D, 128)),
            full((_D, 128)),
            full((1, 128)),
            full((128, _D)),
            full((1, _D)),
            full((1, _D)),
            full((1, 1)),
        ],
        out_specs=pl.BlockSpec((_BLK, 2), lambda n: (n, 0)),
        out_shape=jax.ShapeDtypeStruct((_B // 2, 2), jnp.float32),
    )(u2, i2, w1u, w1i, b1, w2, b2, w3t, b3)


def kernel(inputs, user_emb, item_emb, W1, b1, W2, b2, W3, b3):
    user_idx = inputs[:, 0].reshape(_B // _CHUNK, _CHUNK)
    item_idx = inputs[:, 1].reshape(_B // _CHUNK, _CHUNK)
    u_vec, i_vec = _gather()(user_idx, item_idx,
                             user_emb.astype(jnp.bfloat16),
                             item_emb.astype(jnp.bfloat16))
    u2 = u_vec.reshape(_B // 2, 2 * _D)
    i2 = i_vec.reshape(_B // 2, 2 * _D)
    w1 = W1.astype(jnp.bfloat16)
    out2 = _mlp(u2, i2, w1[:_D], w1[_D:], b1.reshape(1, 128),
                W2.astype(jnp.bfloat16), b2.reshape(1, _D),
                W3.reshape(1, _D), b3.reshape(1, 1))
    return out2.reshape(_B, 1)


# speedup vs baseline: 1.3482x; 1.3482x over previous
import functools

import jax
import jax.numpy as jnp
from jax import lax
from jax.experimental import pallas as pl
from jax.experimental.pallas import tpu as pltpu
from jax.experimental.pallas import tpu_sc as plsc

_B = 16384
_D = 64
_NC = 2
_NS = 16
_NW = _NC * _NS
_BPW = _B // _NW

_FLIGHT = 16
_BLK = 2048


def _fetch_rows(emb_hbm, idx_s, rows_v, sem):
    def burst(ci, _):
        base = ci * _FLIGHT
        vals = idx_s[pl.ds(base, _FLIGHT)]
        copies = []
        for j in range(_FLIGHT):
            copies.append(pltpu.async_copy(
                emb_hbm.at[pl.ds(vals[j], 1)],
                rows_v.at[pl.ds(base + j, 1)], sem))
        for c in copies:
            c.wait()
        return ()
    lax.fori_loop(0, _BPW // _FLIGHT, burst, (), unroll=False)


def _gather_body(uidx_hbm, iidx_hbm, uemb_hbm, iemb_hbm, uout_hbm, iout_hbm,
                 uidx_s, iidx_s, rows_v, sem):
    wid = lax.axis_index("s") * _NC + lax.axis_index("c")
    base = wid * _BPW
    pltpu.sync_copy(uidx_hbm.at[pl.ds(base, _BPW)], uidx_s)
    pltpu.sync_copy(iidx_hbm.at[pl.ds(base, _BPW)], iidx_s)
    _fetch_rows(uemb_hbm, uidx_s, rows_v, sem)
    pltpu.sync_copy(rows_v, uout_hbm.at[pl.ds(base, _BPW)])
    _fetch_rows(iemb_hbm, iidx_s, rows_v, sem)
    pltpu.sync_copy(rows_v, iout_hbm.at[pl.ds(base, _BPW)])


@functools.cache
def _gather():
    return pl.kernel(
        _gather_body,
        out_type=(jax.ShapeDtypeStruct((_B, _D), jnp.float32),
                  jax.ShapeDtypeStruct((_B, _D), jnp.float32)),
        mesh=plsc.VectorSubcoreMesh(core_axis_name="c", subcore_axis_name="s",
                                    num_cores=_NC, num_subcores=_NS),
        scratch_types=[
            pltpu.VMEM((_BPW,), jnp.int32),
            pltpu.VMEM((_BPW,), jnp.int32),
            pltpu.VMEM((_BPW, _D), jnp.float32),
            pltpu.SemaphoreType.DMA,
        ],
    )


def _mlp_body(u_ref, i_ref, w1u_ref, w1i_ref, b1_ref, w2_ref, b2_ref,
              w3t_ref, b3_ref, o_ref):
    u = u_ref[...]
    it = i_ref[...]
    h1 = jnp.dot(u, w1u_ref[...], preferred_element_type=jnp.float32)
    h1 += jnp.dot(it, w1i_ref[...], preferred_element_type=jnp.float32)
    h1 = jnp.maximum(h1 + b1_ref[...], 0.0)
    h2 = jnp.dot(h1, w2_ref[...], preferred_element_type=jnp.float32)
    h2 = jnp.maximum(h2 + b2_ref[...], 0.0)
    logit = jnp.sum(h2 * w3t_ref[...], axis=1, keepdims=True) + b3_ref[...]
    o_ref[...] = 1.0 / (1.0 + jnp.exp(-logit))


def _mlp(u, it, w1u, w1i, b1, w2, b2, w3t, b3):
    full = lambda s: pl.BlockSpec(s, lambda n: (0, 0))
    return pl.pallas_call(
        _mlp_body,
        grid=(_B // _BLK,),
        in_specs=[
            pl.BlockSpec((_BLK, _D), lambda n: (n, 0)),
            pl.BlockSpec((_BLK, _D), lambda n: (n, 0)),
            full((_D, 128)),
            full((_D, 128)),
            full((1, 128)),
            full((128, _D)),
            full((1, _D)),
            full((1, _D)),
            full((1, 1)),
        ],
        out_specs=pl.BlockSpec((_BLK, 1), lambda n: (n, 0)),
        out_shape=jax.ShapeDtypeStruct((_B, 1), jnp.float32),
    )(u, it, w1u, w1i, b1, w2, b2, w3t, b3)


def kernel(inputs, user_emb, item_emb, W1, b1, W2, b2, W3, b3):
    user_idx = inputs[:, 0]
    item_idx = inputs[:, 1]
    u_vec, i_vec = _gather()(user_idx, item_idx, user_emb, item_emb)
    return _mlp(u_vec, i_vec,
                W1[:_D], W1[_D:], b1.reshape(1, 128),
                W2, b2.reshape(1, _D),
                W3.reshape(1, _D), b3.reshape(1, 1))
